# baseline (speedup 1.0000x reference)
import functools
import jax
import jax.numpy as jnp
from jax import lax
from jax.experimental import pallas as pl
from jax.experimental.pallas import tpu as pltpu
from jax.experimental.pallas import tpu_sc as plsc

_PERIOD = 4
_START = 2
_NC = 2
_NS = 16
_NW = _NC * _NS
_RCH = 8
_CCH = 2048
_CO = _CCH // _PERIOD
_NB = 4
_AHEAD = 3


def kernel(x):
    b, n, t = x.shape
    rows = b * n
    k = t // _PERIOD
    rows_per_w = rows // _NW
    col_chunks = t // _CCH
    n_iter = (rows_per_w // _RCH) * col_chunks
    assert n_iter % _NB == 0

    x2 = x.reshape(rows, t)
    mesh = plsc.VectorSubcoreMesh(core_axis_name="c", subcore_axis_name="s")

    @functools.partial(
        pl.kernel,
        mesh=mesh,
        out_type=jax.ShapeDtypeStruct((rows, k), jnp.float32),
        compiler_params=pltpu.CompilerParams(needs_layout_passes=False),
        scratch_types=(
            [pltpu.VMEM((_RCH, _CCH), jnp.float32)] * _NB
            + [pltpu.VMEM((_RCH, _CO), jnp.float32)] * _NB
            + [pltpu.SemaphoreType.DMA] * (2 * _NB)
        ),
    )
    def run(x_hbm, y_hbm, *refs):
        in_bufs = refs[:_NB]
        out_bufs = refs[_NB : 2 * _NB]
        in_sems = refs[2 * _NB : 3 * _NB]
        out_sems = refs[3 * _NB : 4 * _NB]

        cid = lax.axis_index("c")
        sid = lax.axis_index("s")
        wid = sid * _NC + cid
        row0 = wid * rows_per_w

        idx0 = lax.iota(jnp.int32, 16) * _PERIOD + _START

        def chunk_origin(g):
            r = pl.multiple_of(row0 + (g // col_chunks) * _RCH, _RCH)
            c = pl.multiple_of((g % col_chunks) * _CCH, _CCH)
            return r, c

        def start_in(bi, g):
            r, c = chunk_origin(g)
            pltpu.make_async_copy(
                x_hbm.at[pl.ds(r, _RCH), pl.ds(c, _CCH)],
                in_bufs[bi],
                in_sems[bi],
            ).start()

        def wait_in(bi):
            pltpu.make_async_copy(
                x_hbm.at[pl.ds(0, _RCH), pl.ds(0, _CCH)],
                in_bufs[bi],
                in_sems[bi],
            ).wait()

        def start_out(bi, g):
            r, c = chunk_origin(g)
            pltpu.make_async_copy(
                out_bufs[bi],
                y_hbm.at[
                    pl.ds(r, _RCH),
                    pl.ds(pl.multiple_of(c // _PERIOD, _CO), _CO),
                ],
                out_sems[bi],
            ).start()

        def wait_out(bi):
            pltpu.make_async_copy(
                out_bufs[bi],
                y_hbm.at[pl.ds(0, _RCH), pl.ds(0, _CO)],
                out_sems[bi],
            ).wait()

        for p in range(_AHEAD):
            start_in(p, p)

        def step(i, carry):
            for bi in range(_NB):
                g = _NB * i + bi

                @pl.when(g + _AHEAD < n_iter)
                def _():
                    start_in((bi + _AHEAD) % _NB, g + _AHEAD)

                wait_in(bi)

                @pl.when(g >= _NB)
                def _():
                    wait_out(bi)

                for r in range(_RCH):
                    ridx = jnp.full((16,), r, jnp.int32)

                    def cbody(j, c, ridx=ridx, bi=bi, r=r):
                        idx = idx0 + j * (16 * _PERIOD)
                        v = plsc.load_gather(in_bufs[bi], [ridx, idx])
                        out_bufs[bi][r, pl.ds(j * 16, 16)] = v
                        return c

                    lax.fori_loop(0, _CO // 16, cbody, 0, unroll=8)
                start_out(bi, g)
            return carry

        lax.fori_loop(0, n_iter // _NB, step, 0)
        for bi in range(_NB):
            wait_out(bi)

    return run(x2).reshape(b, n, k)


# speedup vs baseline: 1.2213x; 1.2213x over previous
import functools
import jax
import jax.numpy as jnp
from jax import lax
from jax.experimental import pallas as pl
from jax.experimental.pallas import tpu as pltpu
from jax.experimental.pallas import tpu_sc as plsc

_PERIOD = 4
_START = 2
_NC = 2
_NS = 16
_NW = _NC * _NS
_RCH = 8
_CCH = 4096
_CO = _CCH // _PERIOD
_NB = 3
_AHEAD = 2


def kernel(x):
    b, n, t = x.shape
    rows = b * n
    k = t // _PERIOD
    rows_per_w = rows // _NW
    col_chunks = t // _CCH
    n_iter = (rows_per_w // _RCH) * col_chunks
    n_main = (n_iter // _NB) * _NB

    x2 = x.reshape(rows, t)
    mesh = plsc.VectorSubcoreMesh(core_axis_name="c", subcore_axis_name="s")

    @functools.partial(
        pl.kernel,
        mesh=mesh,
        out_type=jax.ShapeDtypeStruct((rows, k), jnp.float32),
        compiler_params=pltpu.CompilerParams(needs_layout_passes=False),
        scratch_types=(
            [pltpu.VMEM((_RCH, _CCH), jnp.float32)] * _NB
            + [pltpu.VMEM((_RCH, _CO), jnp.float32)] * _NB
            + [pltpu.SemaphoreType.DMA] * (2 * _NB)
        ),
    )
    def run(x_hbm, y_hbm, *refs):
        in_bufs = refs[:_NB]
        out_bufs = refs[_NB : 2 * _NB]
        in_sems = refs[2 * _NB : 3 * _NB]
        out_sems = refs[3 * _NB : 4 * _NB]

        cid = lax.axis_index("c")
        sid = lax.axis_index("s")
        wid = sid * _NC + cid
        row0 = wid * rows_per_w

        idx0 = lax.iota(jnp.int32, 16) * _PERIOD + _START

        def chunk_origin(g):
            r = pl.multiple_of(row0 + (g // col_chunks) * _RCH, _RCH)
            c = pl.multiple_of((g % col_chunks) * _CCH, _CCH)
            return r, c

        def start_in(bi, g):
            r, c = chunk_origin(g)
            pltpu.make_async_copy(
                x_hbm.at[pl.ds(r, _RCH), pl.ds(c, _CCH)],
                in_bufs[bi],
                in_sems[bi],
            ).start()

        def wait_in(bi):
            pltpu.make_async_copy(
                x_hbm.at[pl.ds(0, _RCH), pl.ds(0, _CCH)],
                in_bufs[bi],
                in_sems[bi],
            ).wait()

        def start_out(bi, g):
            r, c = chunk_origin(g)
            pltpu.make_async_copy(
                out_bufs[bi],
                y_hbm.at[
                    pl.ds(r, _RCH),
                    pl.ds(pl.multiple_of(c // _PERIOD, _CO), _CO),
                ],
                out_sems[bi],
            ).start()

        def wait_out(bi):
            pltpu.make_async_copy(
                out_bufs[bi],
                y_hbm.at[pl.ds(0, _RCH), pl.ds(0, _CO)],
                out_sems[bi],
            ).wait()

        for p in range(_AHEAD):
            start_in(p, p)

        def step(i, carry, base=0):
            for bi in range(_NB):
                g = base + _NB * i + bi

                @pl.when(g + _AHEAD < n_iter)
                def _():
                    start_in((bi + _AHEAD) % _NB, g + _AHEAD)

                wait_in(bi)

                @pl.when(g >= _NB)
                def _():
                    wait_out(bi)

                for r in range(_RCH):
                    ridx = jnp.full((16,), r, jnp.int32)

                    def cbody(j, c, ridx=ridx, bi=bi, r=r):
                        idx = idx0 + j * (16 * _PERIOD)
                        v = plsc.load_gather(in_bufs[bi], [ridx, idx])
                        out_bufs[bi][r, pl.ds(j * 16, 16)] = v
                        return c

                    lax.fori_loop(0, _CO // 16, cbody, 0, unroll=8)
                start_out(bi, g)
            return carry

        lax.fori_loop(0, n_iter // _NB, step, 0)
        for g in range(n_main, n_iter):
            bi = g % _NB
            wait_in(bi)
            wait_out(bi)
            for r in range(_RCH):
                ridx = jnp.full((16,), r, jnp.int32)

                def cbody(j, c, ridx=ridx, bi=bi, r=r):
                    idx = idx0 + j * (16 * _PERIOD)
                    v = plsc.load_gather(in_bufs[bi], [ridx, idx])
                    out_bufs[bi][r, pl.ds(j * 16, 16)] = v
                    return c

                lax.fori_loop(0, _CO // 16, cbody, 0, unroll=8)
            start_out(bi, g)
        for bi in range(_NB):
            wait_out(bi)

    return run(x2).reshape(b, n, k)


# speedup vs baseline: 1.2663x; 1.0369x over previous
import functools
import jax
import jax.numpy as jnp
from jax import lax
from jax.experimental import pallas as pl
from jax.experimental.pallas import tpu as pltpu
from jax.experimental.pallas import tpu_sc as plsc

_PERIOD = 4
_START = 2
_NC = 2
_NS = 16
_NW = _NC * _NS
_RCH = 8
_CCH = 4096
_CO = _CCH // _PERIOD
_NB = 2
_AHEAD = 1


def kernel(x):
    b, n, t = x.shape
    rows = b * n
    k = t // _PERIOD
    rows_per_w = rows // _NW
    col_chunks = t // _CCH
    n_iter = (rows_per_w // _RCH) * col_chunks
    n_main = (n_iter // _NB) * _NB

    x2 = x.reshape(rows, t)
    mesh = plsc.VectorSubcoreMesh(core_axis_name="c", subcore_axis_name="s")

    @functools.partial(
        pl.kernel,
        mesh=mesh,
        out_type=jax.ShapeDtypeStruct((rows, k), jnp.float32),
        compiler_params=pltpu.CompilerParams(needs_layout_passes=False),
        scratch_types=(
            [pltpu.VMEM((_RCH, _CCH), jnp.float32)] * _NB
            + [pltpu.VMEM((_RCH, _CO), jnp.float32)] * _NB
            + [pltpu.SemaphoreType.DMA] * (2 * _NB)
        ),
    )
    def run(x_hbm, y_hbm, *refs):
        in_bufs = refs[:_NB]
        out_bufs = refs[_NB : 2 * _NB]
        in_sems = refs[2 * _NB : 3 * _NB]
        out_sems = refs[3 * _NB : 4 * _NB]

        cid = lax.axis_index("c")
        sid = lax.axis_index("s")
        wid = sid * _NC + cid
        row0 = wid * rows_per_w

        idx0 = lax.iota(jnp.int32, 16) * _PERIOD + _START

        def chunk_origin(g):
            r = pl.multiple_of(row0 + (g // col_chunks) * _RCH, _RCH)
            c = pl.multiple_of((g % col_chunks) * _CCH, _CCH)
            return r, c

        def start_in(bi, g):
            r, c = chunk_origin(g)
            pltpu.make_async_copy(
                x_hbm.at[pl.ds(r, _RCH), pl.ds(c, _CCH)],
                in_bufs[bi],
                in_sems[bi],
            ).start()

        def wait_in(bi):
            pltpu.make_async_copy(
                x_hbm.at[pl.ds(0, _RCH), pl.ds(0, _CCH)],
                in_bufs[bi],
                in_sems[bi],
            ).wait()

        def start_out(bi, g):
            r, c = chunk_origin(g)
            pltpu.make_async_copy(
                out_bufs[bi],
                y_hbm.at[
                    pl.ds(r, _RCH),
                    pl.ds(pl.multiple_of(c // _PERIOD, _CO), _CO),
                ],
                out_sems[bi],
            ).start()

        def wait_out(bi):
            pltpu.make_async_copy(
                out_bufs[bi],
                y_hbm.at[pl.ds(0, _RCH), pl.ds(0, _CO)],
                out_sems[bi],
            ).wait()

        for p in range(_AHEAD):
            start_in(p, p)

        def step(i, carry, base=0):
            for bi in range(_NB):
                g = base + _NB * i + bi

                @pl.when(g + _AHEAD < n_iter)
                def _():
                    start_in((bi + _AHEAD) % _NB, g + _AHEAD)

                wait_in(bi)

                @pl.when(g >= _NB)
                def _():
                    wait_out(bi)

                for r in range(_RCH):
                    ridx = jnp.full((16,), r, jnp.int32)

                    def cbody(j, c, ridx=ridx, bi=bi, r=r):
                        idx = idx0 + j * (16 * _PERIOD)
                        v = plsc.load_gather(in_bufs[bi], [ridx, idx])
                        out_bufs[bi][r, pl.ds(j * 16, 16)] = v
                        return c

                    lax.fori_loop(0, _CO // 16, cbody, 0, unroll=8)
                start_out(bi, g)
            return carry

        lax.fori_loop(0, n_iter // _NB, step, 0)
        for g in range(n_main, n_iter):
            bi = g % _NB
            wait_in(bi)
            wait_out(bi)
            for r in range(_RCH):
                ridx = jnp.full((16,), r, jnp.int32)

                def cbody(j, c, ridx=ridx, bi=bi, r=r):
                    idx = idx0 + j * (16 * _PERIOD)
                    v = plsc.load_gather(in_bufs[bi], [ridx, idx])
                    out_bufs[bi][r, pl.ds(j * 16, 16)] = v
                    return c

                lax.fori_loop(0, _CO // 16, cbody, 0, unroll=8)
            start_out(bi, g)
        for bi in range(_NB):
            wait_out(bi)

    return run(x2).reshape(b, n, k)
